# baseline (speedup 1.0000x reference)
import functools

import jax
import jax.numpy as jnp
from jax.experimental import pallas as pl

_L = 1024
_NPAD = 104


def _encode_kernel(a_ref, g_ref, gt_ref, o_ref):
    a = a_ref[...]
    ax1 = a[0:1, :]
    ay1 = a[1:2, :]
    ax2 = a[2:3, :]
    ay2 = a[3:4, :]
    aw = a[4:5, :]
    ah = a[5:6, :]
    acx = a[6:7, :]
    acy = a[7:8, :]

    g = g_ref[0]
    bx1 = g[:, 0:1]
    by1 = g[:, 1:2]
    bx2 = g[:, 2:3]
    by2 = g[:, 3:4]
    barea = g[:, 4:5]

    iw = jnp.maximum(jnp.minimum(ax2, bx2) - jnp.maximum(ax1, bx1), 0.0)
    ih = jnp.maximum(jnp.minimum(ay2, by2) - jnp.maximum(ay1, by1), 0.0)
    inter = iw * ih
    area_a = aw * ah
    union = area_a + barea - inter
    iou = inter / jnp.maximum(union, 1e-8)

    max_iou = jnp.max(iou, axis=0, keepdims=True)
    sub = jax.lax.broadcasted_iota(jnp.int32, iou.shape, 0)
    midx = jnp.min(jnp.where(iou == max_iou, sub, _NPAD), axis=0,
                   keepdims=True)
    onehot = (sub == midx).astype(jnp.bfloat16)

    gtr = gt_ref[0]
    hi_f = jax.lax.bitcast_convert_type(
        jax.lax.bitcast_convert_type(gtr, jnp.uint32) & jnp.uint32(0xFFFF0000),
        jnp.float32)
    r1 = gtr - hi_f
    mid_f = jax.lax.bitcast_convert_type(
        jax.lax.bitcast_convert_type(r1, jnp.uint32) & jnp.uint32(0xFFFF0000),
        jnp.float32)
    lo = r1 - mid_f
    gt24 = jnp.concatenate(
        [hi_f.astype(jnp.bfloat16), mid_f.astype(jnp.bfloat16),
         lo.astype(jnp.bfloat16)], axis=0)
    g24 = jax.lax.dot_general(
        gt24, onehot, (((1,), (0,)), ((), ())),
        preferred_element_type=jnp.float32)
    g8 = (g24[0:8, :] + g24[8:16, :]) + g24[16:24, :]
    gx1 = g8[0:1, :]
    gy1 = g8[1:2, :]
    gx2 = g8[2:3, :]
    gy2 = g8[3:4, :]
    gcls = g8[5:6, :]

    gw = gx2 - gx1
    gh = gy2 - gy1
    gcx = gx1 + gw * 0.5
    gcy = gy1 + gh * 0.5

    tx = ((gcx - acx) / aw) / 0.1
    ty = ((gcy - acy) / ah) / 0.1
    tw = jnp.log(gw / aw) / 0.2
    th = jnp.log(gh / ah) / 0.2

    pos = max_iou >= 0.5
    ign = jnp.logical_and(max_iou >= 0.4, max_iou < 0.5)
    cls = jnp.where(pos, gcls, -1.0)
    cls = jnp.where(ign, -2.0, cls)

    out = jnp.concatenate(
        [tx, ty, tw, th, cls, cls, cls, cls], axis=0)
    out = jnp.where(jnp.isnan(out), -2.0, out)
    o_ref[0] = out


@functools.partial(jax.jit, static_argnums=())
def kernel(images, gt_boxes, gt_classes, anchor_boxes):
    del images
    B, N = gt_classes.shape
    A = anchor_boxes.shape[0]
    G = -(-A // _L)
    A_pad = G * _L

    x1, y1, x2, y2 = (anchor_boxes[:, i] for i in range(4))
    aw = x2 - x1
    ah = y2 - y1
    acx = x1 + aw * 0.5
    acy = y1 + ah * 0.5
    aT = jnp.stack([x1, y1, x2, y2, aw, ah, acx, acy], axis=0)
    pad = jnp.broadcast_to(
        jnp.asarray([0.0, 0.0, 1.0, 1.0, 1.0, 1.0, 0.5, 0.5],
                    jnp.float32)[:, None], (8, A_pad - A))
    aT = jnp.concatenate([aT, pad], axis=1)

    gx1, gy1, gx2, gy2 = (gt_boxes[..., i] for i in range(4))
    area = (gx2 - gx1) * (gy2 - gy1)
    zeros = jnp.zeros_like(gx1)
    cols = jnp.stack([gx1, gy1, gx2, gy2, area, gt_classes, zeros, zeros],
                     axis=-1)
    gt_cols = jnp.pad(cols, ((0, 0), (0, _NPAD - N), (0, 0)))
    gt_rowsT = jnp.transpose(gt_cols, (0, 2, 1))

    out = pl.pallas_call(
        _encode_kernel,
        grid=(G, B),
        in_specs=[
            pl.BlockSpec((8, _L), lambda g, b: (0, g)),
            pl.BlockSpec((1, _NPAD, 8), lambda g, b: (b, 0, 0)),
            pl.BlockSpec((1, 8, _NPAD), lambda g, b: (b, 0, 0)),
        ],
        out_specs=pl.BlockSpec((1, 8, _L), lambda g, b: (b, 0, g)),
        out_shape=jax.ShapeDtypeStruct((B, 8, A_pad), jnp.float32),
    )(aT, gt_cols, gt_rowsT)

    box = jnp.transpose(out[:, 0:4, :A], (0, 2, 1))
    cls = out[:, 4, :A]
    return box, cls


# speedup vs baseline: 1.5219x; 1.5219x over previous
import functools

import jax
import jax.numpy as jnp
from jax.experimental import pallas as pl

_L = 2048
_NPAD = 104


def _encode_kernel(a_ref, g_ref, gt_ref, o_ref):
    a = a_ref[...]
    ax1 = a[0:1, :]
    ay1 = a[1:2, :]
    ax2 = a[2:3, :]
    ay2 = a[3:4, :]
    aw = a[4:5, :]
    ah = a[5:6, :]
    acx = a[6:7, :]
    acy = a[7:8, :]

    g = g_ref[0]
    bx1 = g[:, 0:1]
    by1 = g[:, 1:2]
    bx2 = g[:, 2:3]
    by2 = g[:, 3:4]
    barea = g[:, 4:5]

    iw = jnp.maximum(jnp.minimum(ax2, bx2) - jnp.maximum(ax1, bx1), 0.0)
    ih = jnp.maximum(jnp.minimum(ay2, by2) - jnp.maximum(ay1, by1), 0.0)
    inter = iw * ih
    area_a = aw * ah
    union = area_a + barea - inter
    iou = inter / union

    max_iou = jnp.max(iou, axis=0, keepdims=True)
    sub = jax.lax.broadcasted_iota(jnp.int32, iou.shape, 0)
    midx = jnp.min(jnp.where(iou == max_iou, sub, _NPAD), axis=0,
                   keepdims=True)
    onehot = (sub == midx).astype(jnp.bfloat16)

    gtr = gt_ref[0]
    hi_f = jax.lax.bitcast_convert_type(
        jax.lax.bitcast_convert_type(gtr, jnp.uint32) & jnp.uint32(0xFFFF0000),
        jnp.float32)
    r1 = gtr - hi_f
    mid_f = jax.lax.bitcast_convert_type(
        jax.lax.bitcast_convert_type(r1, jnp.uint32) & jnp.uint32(0xFFFF0000),
        jnp.float32)
    lo = r1 - mid_f
    gt24 = jnp.concatenate(
        [hi_f.astype(jnp.bfloat16), mid_f.astype(jnp.bfloat16),
         lo.astype(jnp.bfloat16)], axis=0)
    g24 = jax.lax.dot_general(
        gt24, onehot, (((1,), (0,)), ((), ())),
        preferred_element_type=jnp.float32)
    g8 = (g24[0:8, :] + g24[8:16, :]) + g24[16:24, :]
    gx1 = g8[0:1, :]
    gy1 = g8[1:2, :]
    gx2 = g8[2:3, :]
    gy2 = g8[3:4, :]
    gcls = g8[5:6, :]

    gw = gx2 - gx1
    gh = gy2 - gy1
    gcx = gx1 + gw * 0.5
    gcy = gy1 + gh * 0.5

    tx = ((gcx - acx) / aw) / 0.1
    ty = ((gcy - acy) / ah) / 0.1
    tw = jnp.log(gw / aw) / 0.2
    th = jnp.log(gh / ah) / 0.2

    pos = max_iou >= 0.5
    ign = jnp.logical_and(max_iou >= 0.4, max_iou < 0.5)
    cls = jnp.where(pos, gcls, -1.0)
    cls = jnp.where(ign, -2.0, cls)

    out = jnp.concatenate(
        [tx, ty, tw, th, cls, cls, cls, cls], axis=0)
    out = jnp.where(jnp.isnan(out), -2.0, out)
    o_ref[0] = out


@functools.partial(jax.jit, static_argnums=())
def kernel(images, gt_boxes, gt_classes, anchor_boxes):
    del images
    B, N = gt_classes.shape
    A = anchor_boxes.shape[0]
    G = -(-A // _L)
    A_pad = G * _L

    x1, y1, x2, y2 = (anchor_boxes[:, i] for i in range(4))
    aw = x2 - x1
    ah = y2 - y1
    acx = x1 + aw * 0.5
    acy = y1 + ah * 0.5
    aT = jnp.stack([x1, y1, x2, y2, aw, ah, acx, acy], axis=0)
    pad = jnp.broadcast_to(
        jnp.asarray([0.0, 0.0, 1.0, 1.0, 1.0, 1.0, 0.5, 0.5],
                    jnp.float32)[:, None], (8, A_pad - A))
    aT = jnp.concatenate([aT, pad], axis=1)

    gx1, gy1, gx2, gy2 = (gt_boxes[..., i] for i in range(4))
    area = (gx2 - gx1) * (gy2 - gy1)
    zeros = jnp.zeros_like(gx1)
    cols = jnp.stack([gx1, gy1, gx2, gy2, area, gt_classes, zeros, zeros],
                     axis=-1)
    gt_cols = jnp.pad(cols, ((0, 0), (0, _NPAD - N), (0, 0)))
    gt_rowsT = jnp.transpose(gt_cols, (0, 2, 1))

    out = pl.pallas_call(
        _encode_kernel,
        grid=(G, B),
        in_specs=[
            pl.BlockSpec((8, _L), lambda g, b: (0, g)),
            pl.BlockSpec((1, _NPAD, 8), lambda g, b: (b, 0, 0)),
            pl.BlockSpec((1, 8, _NPAD), lambda g, b: (b, 0, 0)),
        ],
        out_specs=pl.BlockSpec((1, 8, _L), lambda g, b: (b, 0, g)),
        out_shape=jax.ShapeDtypeStruct((B, 8, A_pad), jnp.float32),
    )(aT, gt_cols, gt_rowsT)

    box = jnp.transpose(out[:, 0:4, :A], (0, 2, 1))
    cls = out[:, 4, :A]
    return box, cls
